# baseline (speedup 1.0000x reference)
import functools

import jax
import jax.numpy as jnp
from jax import lax
from jax.experimental import pallas as pl
from jax.experimental.pallas import tpu as pltpu
from jax.experimental.pallas import tpu_sc as plsc

VOCAB = 1000000
EMB = 64
B = 16384
L = 16
NC = 2
NS = 16
NW = NC * NS
BPW = B // NW
NCHUNK = 4
CHUNK = BPW // NCHUNK
W = 2 * EMB

_mesh = plsc.VectorSubcoreMesh(core_axis_name="c", subcore_axis_name="s")


@functools.partial(
    pl.kernel,
    mesh=_mesh,
    out_type=(
        jax.ShapeDtypeStruct((B,), jnp.float32),
        jax.ShapeDtypeStruct((B,), jnp.float32),
    ),
    scratch_types=[
        pltpu.VMEM((NCHUNK, CHUNK), jnp.int32),
        pltpu.VMEM((NCHUNK, CHUNK), jnp.int32),
        pltpu.VMEM((NCHUNK, CHUNK), jnp.int32),
        pltpu.VMEM((BPW,), jnp.int32),
        pltpu.VMEM((BPW,), jnp.int32),
        pltpu.VMEM((BPW,), jnp.int32),
        pltpu.VMEM((CHUNK, W), jnp.float32),
        pltpu.VMEM((CHUNK, W), jnp.float32),
        pltpu.VMEM((CHUNK, W), jnp.float32),
        pltpu.VMEM((CHUNK, W), jnp.float32),
        pltpu.VMEM((CHUNK, W), jnp.float32),
        pltpu.VMEM((CHUNK, W), jnp.float32),
        pltpu.VMEM((BPW,), jnp.float32),
        pltpu.VMEM((BPW,), jnp.float32),
        pltpu.SemaphoreType.DMA,
        pltpu.SemaphoreType.DMA,
    ],
)
def _sc_dots(tw_hbm, cw_hbm, ng_hbm, oft_hbm, ofc_hbm, ofn_hbm,
             temb_hbm, cemb_hbm,
             pos_hbm, neg_hbm,
             tw_v, cw_v, ng_v, oft_v, ofc_v, ofn_v,
             tgt0, tgt1, ctx0, ctx1, ngr0, ngr1,
             pd_v, nd_v, sem0, sem1):
    wid = lax.axis_index("s") * NC + lax.axis_index("c")
    pltpu.sync_copy(tw_hbm.at[wid], tw_v)
    pltpu.sync_copy(cw_hbm.at[wid], cw_v)
    pltpu.sync_copy(ng_hbm.at[wid], ng_v)
    pltpu.sync_copy(oft_hbm.at[wid], oft_v)
    pltpu.sync_copy(ofc_hbm.at[wid], ofc_v)
    pltpu.sync_copy(ofn_hbm.at[wid], ofn_v)

    tgt_b = (tgt0, tgt1)
    ctx_b = (ctx0, ctx1)
    ngr_b = (ngr0, ngr1)
    sems = (sem0, sem1)

    def fire(q):
        par = q % 2
        return (
            pltpu.async_copy(cemb_hbm.at[cw_v.at[q]], ctx_b[par], sems[par]),
            pltpu.async_copy(temb_hbm.at[tw_v.at[q]], tgt_b[par], sems[par]),
            pltpu.async_copy(temb_hbm.at[ng_v.at[q]], ngr_b[par], sems[par]),
        )

    lane = lax.iota(jnp.int32, L)
    perms = [lane ^ s for s in (1, 2, 4, 8)]
    dnums = lax.GatherDimensionNumbers(
        offset_dims=(), collapsed_slice_dims=(0,), start_index_map=(0,))

    def lane_sum(v):
        for p in perms:
            v = v + lax.gather(
                v, p[:, None], dnums, slice_sizes=(1,),
                mode=lax.GatherScatterMode.PROMISE_IN_BOUNDS)
        return v

    def compute(q):
        par = q % 2
        cb, tb, nb = ctx_b[par], tgt_b[par], ngr_b[par]

        def body(g, carry):
            acc_p = jnp.zeros((L,), jnp.float32)
            acc_n = jnp.zeros((L,), jnp.float32)
            ovc = ofc_v[pl.ds(q * CHUNK + g * L, L)]
            ovt = oft_v[pl.ds(q * CHUNK + g * L, L)]
            ovn = ofn_v[pl.ds(q * CHUNK + g * L, L)]
            for k in range(L):
                rb = g * L + k
                oc = ovc[k]
                ot = ovt[k]
                on = ovn[k]
                c0 = cb[rb, pl.ds(oc, L)]
                c1 = cb[rb, pl.ds(oc + L, L)]
                c2 = cb[rb, pl.ds(oc + 2 * L, L)]
                c3 = cb[rb, pl.ds(oc + 3 * L, L)]
                pp = tb[rb, pl.ds(ot, L)] * c0
                pp = pp + tb[rb, pl.ds(ot + L, L)] * c1
                pp = pp + tb[rb, pl.ds(ot + 2 * L, L)] * c2
                pp = pp + tb[rb, pl.ds(ot + 3 * L, L)] * c3
                nn = nb[rb, pl.ds(on, L)] * c0
                nn = nn + nb[rb, pl.ds(on + L, L)] * c1
                nn = nn + nb[rb, pl.ds(on + 2 * L, L)] * c2
                nn = nn + nb[rb, pl.ds(on + 3 * L, L)] * c3
                acc_p = jnp.where(lane == k, lane_sum(pp), acc_p)
                acc_n = jnp.where(lane == k, lane_sum(nn), acc_n)
            pd_v[pl.ds(q * CHUNK + g * L, L)] = acc_p
            nd_v[pl.ds(q * CHUNK + g * L, L)] = acc_n
            return carry

        lax.fori_loop(0, CHUNK // L, body, 0)

    pending = {0: fire(0)}
    for q in range(NCHUNK):
        if q + 1 < NCHUNK:
            pending[q + 1] = fire(q + 1)
        for c in pending.pop(q):
            c.wait()
        compute(q)

    base = wid * BPW
    pltpu.sync_copy(pd_v, pos_hbm.at[pl.ds(base, BPW)])
    pltpu.sync_copy(nd_v, neg_hbm.at[pl.ds(base, BPW)])


def _loss_body(pos_ref, neg_ref, out_ref):
    p = pos_ref[...]
    n = -neg_ref[...]
    lp = jnp.minimum(p, 0.0) - jnp.log(1.0 + jnp.exp(-jnp.abs(p)))
    ln = jnp.minimum(n, 0.0) - jnp.log(1.0 + jnp.exp(-jnp.abs(n)))
    out_ref[0] = -(jnp.sum(lp) + jnp.sum(ln))


_loss = pl.pallas_call(
    _loss_body,
    out_shape=jax.ShapeDtypeStruct((1,), jnp.float32),
    in_specs=[
        pl.BlockSpec(memory_space=pltpu.VMEM),
        pl.BlockSpec(memory_space=pltpu.VMEM),
    ],
    out_specs=pl.BlockSpec(memory_space=pltpu.SMEM),
)


def kernel(target_word, context_word, negative_example, target_emb, context_emb):
    tw = target_word.astype(jnp.int32)
    cw = context_word.astype(jnp.int32)
    ng = negative_example.astype(jnp.int32)
    t2 = target_emb.reshape(VOCAB // 2, W)
    c2 = context_emb.reshape(VOCAB // 2, W)
    twh = (tw >> 1).reshape(NW, NCHUNK, CHUNK)
    cwh = (cw >> 1).reshape(NW, NCHUNK, CHUNK)
    ngh = (ng >> 1).reshape(NW, NCHUNK, CHUNK)
    oft = ((tw & 1) << 6).reshape(NW, BPW)
    ofc = ((cw & 1) << 6).reshape(NW, BPW)
    ofn = ((ng & 1) << 6).reshape(NW, BPW)
    pos, neg = _sc_dots(twh, cwh, ngh, oft, ofc, ofn, t2, c2)
    loss = _loss(pos.reshape(128, 128), neg.reshape(128, 128))
    return loss[0]


# speedup vs baseline: 2.3949x; 2.3949x over previous
import functools

import jax
import jax.numpy as jnp
from jax import lax
from jax.experimental import pallas as pl
from jax.experimental.pallas import tpu as pltpu
from jax.experimental.pallas import tpu_sc as plsc

VOCAB = 1000000
EMB = 64
B = 16384
L = 16
NC = 2
NS = 16
NW = NC * NS
BPW = B // NW
NG = BPW // L
SUB = 8
NSLAB = VOCAB // SUB

_mesh = plsc.VectorSubcoreMesh(core_axis_name="c", subcore_axis_name="s")


@functools.partial(
    pl.kernel,
    mesh=_mesh,
    out_type=(
        jax.ShapeDtypeStruct((B,), jnp.float32),
        jax.ShapeDtypeStruct((B,), jnp.float32),
    ),
    scratch_types=[
        pltpu.VMEM((BPW,), jnp.int32),
        pltpu.VMEM((BPW,), jnp.int32),
        pltpu.VMEM((BPW,), jnp.int32),
        pltpu.VMEM((2, SUB, EMB), jnp.float32),
        pltpu.VMEM((2, SUB, EMB), jnp.float32),
        pltpu.VMEM((2, SUB, EMB), jnp.float32),
        pltpu.VMEM((2, SUB, EMB), jnp.float32),
        pltpu.VMEM((2, SUB, EMB), jnp.float32),
        pltpu.VMEM((2, SUB, EMB), jnp.float32),
        pltpu.VMEM((BPW,), jnp.float32),
        pltpu.VMEM((BPW,), jnp.float32),
        pltpu.SemaphoreType.DMA,
        pltpu.SemaphoreType.DMA,
    ],
)
def _sc_dots(tw_hbm, cw_hbm, ng_hbm, temb_hbm, cemb_hbm,
             pos_hbm, neg_hbm,
             tw_v, cw_v, ng_v,
             tgt0, tgt1, ctx0, ctx1, ngr0, ngr1,
             pd_v, nd_v, sem0, sem1):
    wid = lax.axis_index("s") * NC + lax.axis_index("c")
    pltpu.sync_copy(tw_hbm.at[wid], tw_v)
    pltpu.sync_copy(cw_hbm.at[wid], cw_v)
    pltpu.sync_copy(ng_hbm.at[wid], ng_v)

    tgt_b = (tgt0, tgt1)
    ctx_b = (ctx0, ctx1)
    ngr_b = (ngr0, ngr1)
    sems = (sem0, sem1)

    def fire(g, par):
        ivt = tw_v[pl.ds(g * L, L)]
        ivc = cw_v[pl.ds(g * L, L)]
        ivn = ng_v[pl.ds(g * L, L)]
        for k in range(L):
            it = ivt[k]
            ic = ivc[k]
            iq = ivn[k]
            dst = (k // SUB, k % SUB)
            pltpu.async_copy(temb_hbm.at[it >> 3, it & 7],
                             tgt_b[par].at[dst[0], dst[1]], sems[par])
            pltpu.async_copy(cemb_hbm.at[ic >> 3, ic & 7],
                             ctx_b[par].at[dst[0], dst[1]], sems[par])
            pltpu.async_copy(temb_hbm.at[iq >> 3, iq & 7],
                             ngr_b[par].at[dst[0], dst[1]], sems[par])

    def drain(par):
        pltpu.make_async_copy(temb_hbm.at[pl.ds(0, 2)], tgt_b[par],
                              sems[par]).wait()
        pltpu.make_async_copy(cemb_hbm.at[pl.ds(0, 2)], ctx_b[par],
                              sems[par]).wait()
        pltpu.make_async_copy(temb_hbm.at[pl.ds(0, 2)], ngr_b[par],
                              sems[par]).wait()

    lane = lax.iota(jnp.int32, L)
    perms = [lane ^ s for s in (1, 2, 4, 8)]
    dnums = lax.GatherDimensionNumbers(
        offset_dims=(), collapsed_slice_dims=(0,), start_index_map=(0,))

    def lane_sum(v):
        for p in perms:
            v = v + lax.gather(
                v, p[:, None], dnums, slice_sizes=(1,),
                mode=lax.GatherScatterMode.PROMISE_IN_BOUNDS)
        return v

    def compute(g, par):
        cb, tb, nb = ctx_b[par], tgt_b[par], ngr_b[par]
        acc_p = jnp.zeros((L,), jnp.float32)
        acc_n = jnp.zeros((L,), jnp.float32)
        for k in range(L):
            a, b = k // SUB, k % SUB
            c0 = cb[a, b, pl.ds(0, L)]
            c1 = cb[a, b, pl.ds(L, L)]
            c2 = cb[a, b, pl.ds(2 * L, L)]
            c3 = cb[a, b, pl.ds(3 * L, L)]
            pp = tb[a, b, pl.ds(0, L)] * c0
            pp = pp + tb[a, b, pl.ds(L, L)] * c1
            pp = pp + tb[a, b, pl.ds(2 * L, L)] * c2
            pp = pp + tb[a, b, pl.ds(3 * L, L)] * c3
            nn = nb[a, b, pl.ds(0, L)] * c0
            nn = nn + nb[a, b, pl.ds(L, L)] * c1
            nn = nn + nb[a, b, pl.ds(2 * L, L)] * c2
            nn = nn + nb[a, b, pl.ds(3 * L, L)] * c3
            acc_p = jnp.where(lane == k, lane_sum(pp), acc_p)
            acc_n = jnp.where(lane == k, lane_sum(nn), acc_n)
        pd_v[pl.ds(g * L, L)] = acc_p
        nd_v[pl.ds(g * L, L)] = acc_n

    fire(0, 0)

    def step(s, carry):
        g0 = 2 * s
        fire(g0 + 1, 1)
        drain(0)
        compute(g0, 0)

        @pl.when(s < NG // 2 - 1)
        def _():
            fire(g0 + 2, 0)

        drain(1)
        compute(g0 + 1, 1)
        return carry

    lax.fori_loop(0, NG // 2, step, 0)

    base = wid * BPW
    pltpu.sync_copy(pd_v, pos_hbm.at[pl.ds(base, BPW)])
    pltpu.sync_copy(nd_v, neg_hbm.at[pl.ds(base, BPW)])


def _loss_body(pos_ref, neg_ref, out_ref):
    p = pos_ref[...]
    n = -neg_ref[...]
    lp = jnp.minimum(p, 0.0) - jnp.log(1.0 + jnp.exp(-jnp.abs(p)))
    ln = jnp.minimum(n, 0.0) - jnp.log(1.0 + jnp.exp(-jnp.abs(n)))
    out_ref[0] = -(jnp.sum(lp) + jnp.sum(ln))


_loss = pl.pallas_call(
    _loss_body,
    out_shape=jax.ShapeDtypeStruct((1,), jnp.float32),
    in_specs=[
        pl.BlockSpec(memory_space=pltpu.VMEM),
        pl.BlockSpec(memory_space=pltpu.VMEM),
    ],
    out_specs=pl.BlockSpec(memory_space=pltpu.SMEM),
)


def kernel(target_word, context_word, negative_example, target_emb, context_emb):
    tw = target_word.astype(jnp.int32).reshape(NW, BPW)
    cw = context_word.astype(jnp.int32).reshape(NW, BPW)
    ng = negative_example.astype(jnp.int32).reshape(NW, BPW)
    t3 = target_emb.reshape(NSLAB, SUB, EMB)
    c3 = context_emb.reshape(NSLAB, SUB, EMB)
    pos, neg = _sc_dots(tw, cw, ng, t3, c3)
    loss = _loss(pos.reshape(128, 128), neg.reshape(128, 128))
    return loss[0]
